# baseline (speedup 1.0000x reference)
import jax
import jax.numpy as jnp
import numpy as np
from jax import lax
from jax.experimental import pallas as pl
from jax.experimental.pallas import tpu as pltpu
from jax.experimental.pallas import tpu_sc as plsc

H = 8
DH = 32
NC = 2
NS = 16
CH = 40
CHB = 80
GB = 2


def _nodeproj_body(x_ref, w_ref, b_ref, q_ref, k_ref, v_ref):
    y = jnp.dot(x_ref[...].astype(jnp.bfloat16), w_ref[...],
                preferred_element_type=jnp.float32)
    y = y + b_ref[...]
    yb = y.astype(jnp.bfloat16)
    q_ref[...] = yb[:, 0:256]
    k_ref[...] = yb[:, 256:512]
    v_ref[...] = yb[:, 512:768]


def _tc_nodeproj(x, wqkv, bqkv, bn):
    n, d = x.shape
    grid = (n // bn,)
    return pl.pallas_call(
        _nodeproj_body,
        grid=grid,
        in_specs=[
            pl.BlockSpec((bn, d), lambda i: (i, 0)),
            pl.BlockSpec((d, 768), lambda i: (0, 0)),
            pl.BlockSpec((1, 768), lambda i: (0, 0)),
        ],
        out_specs=[
            pl.BlockSpec((bn, 256), lambda i: (i, 0)),
            pl.BlockSpec((bn, 256), lambda i: (i, 0)),
            pl.BlockSpec((bn, 256), lambda i: (i, 0)),
        ],
        out_shape=[
            jax.ShapeDtypeStruct((n, 256), jnp.bfloat16),
            jax.ShapeDtypeStruct((n, 256), jnp.bfloat16),
            jax.ShapeDtypeStruct((n, 256), jnp.bfloat16),
        ],
    )(x, wqkv, bqkv)


def _edges_body(ks_ref, qd_ref, ea_ref, vs_ref, ww_ref, wb_ref, bw_ref,
                bb_ref, aw_ref, exp_ref, woe_ref, boe_ref,
                pay_ref, ee_ref, st_ref, acc_sum, acc_sq):
    i = pl.program_id(0)
    ea = ea_ref[...]
    ea_bf = ea.astype(jnp.bfloat16)
    ew = jnp.dot(ea_bf, ww_ref[...], preferred_element_type=jnp.float32) + bw_ref[...]
    eb = jnp.dot(ea_bf, wb_ref[...], preferred_element_type=jnp.float32) + bb_ref[...]
    s = (ks_ref[...].astype(jnp.float32)
         + qd_ref[...].astype(jnp.float32)) * ew
    s = jnp.sign(s) * jnp.sqrt(jnp.abs(s))
    e_t = jnp.maximum(s + eb, 0.0)
    e_bf = e_t.astype(jnp.bfloat16)
    att = jnp.dot(e_bf, aw_ref[...], preferred_element_type=jnp.float32)
    att = jnp.clip(att, -5.0, 5.0)
    xa = jnp.exp(att)
    p256 = jnp.dot(xa, exp_ref[...], preferred_element_type=jnp.float32)
    pay_ref[:, 0:256] = vs_ref[...].astype(jnp.float32) * p256
    pay_ref[:, 256:512] = e_t * p256
    pay_ref[:, 512:528] = xa
    pay_ref[:, 528:640] = jnp.zeros((ea.shape[0], 112), jnp.float32)
    ee = ea + jnp.dot(e_bf, woe_ref[...],
                      preferred_element_type=jnp.float32) + boe_ref[...]
    ee_ref[...] = ee.astype(jnp.bfloat16)

    @pl.when(i == 0)
    def _():
        acc_sum[...] = jnp.zeros_like(acc_sum)
        acc_sq[...] = jnp.zeros_like(acc_sq)

    acc_sum[...] += jnp.sum(ee, axis=0, keepdims=True)
    acc_sq[...] += jnp.sum(ee * ee, axis=0, keepdims=True)

    @pl.when(i == pl.num_programs(0) - 1)
    def _():
        st_ref[0:1, :] = acc_sum[...]
        st_ref[1:2, :] = acc_sq[...]


def _tc_edges(ks, qd, edge_attr, vs, we_w, we_b, be_w, be_b, aw2, exp16,
              woe, boe, be_blk):
    e, d = edge_attr.shape
    grid = (e // be_blk,)
    return pl.pallas_call(
        _edges_body,
        grid=grid,
        in_specs=[
            pl.BlockSpec((be_blk, 256), lambda i: (i, 0)),
            pl.BlockSpec((be_blk, 256), lambda i: (i, 0)),
            pl.BlockSpec((be_blk, d), lambda i: (i, 0)),
            pl.BlockSpec((be_blk, 256), lambda i: (i, 0)),
            pl.BlockSpec((d, 256), lambda i: (0, 0)),
            pl.BlockSpec((d, 256), lambda i: (0, 0)),
            pl.BlockSpec((1, 256), lambda i: (0, 0)),
            pl.BlockSpec((1, 256), lambda i: (0, 0)),
            pl.BlockSpec((256, 16), lambda i: (0, 0)),
            pl.BlockSpec((16, 256), lambda i: (0, 0)),
            pl.BlockSpec((256, 256), lambda i: (0, 0)),
            pl.BlockSpec((1, 256), lambda i: (0, 0)),
        ],
        out_specs=[
            pl.BlockSpec((be_blk, 640), lambda i: (i, 0)),
            pl.BlockSpec((be_blk, 256), lambda i: (i, 0)),
            pl.BlockSpec((8, 256), lambda i: (0, 0)),
        ],
        out_shape=[
            jax.ShapeDtypeStruct((e, 640), jnp.float32),
            jax.ShapeDtypeStruct((e, 256), jnp.bfloat16),
            jax.ShapeDtypeStruct((8, 256), jnp.float32),
        ],
        scratch_shapes=[
            pltpu.VMEM((1, 256), jnp.float32),
            pltpu.VMEM((1, 256), jnp.float32),
        ],
    )(ks, qd, edge_attr, vs, we_w, we_b, be_w, be_b, aw2, exp16, woe, boe)


def _nodes1_body(acc_ref, op_ref, x_ref, ver_ref, exp_ref, woh_ref, boh_ref,
                 dc0_ref, dc1_ref, h_ref, st_ref, acc_sum, acc_sq):
    i = pl.program_id(0)
    s16 = acc_ref[:, 512:528] + op_ref[:, 0:16]
    rs16 = 1.0 / (s16 + 1e-30)
    rs256 = jnp.dot(rs16, exp_ref[...], preferred_element_type=jnp.float32)
    ld = jnp.log(s16[:, 8:9] + 1.0)
    wv = (acc_ref[:, 0:256]
          + jnp.dot(acc_ref[:, 256:512].astype(jnp.bfloat16), ver_ref[...],
                    preferred_element_type=jnp.float32))
    wv = wv * rs256
    hh = wv * dc0_ref[...] + (wv * ld) * dc1_ref[...]
    h = x_ref[...] + jnp.dot(hh.astype(jnp.bfloat16), woh_ref[...],
                             preferred_element_type=jnp.float32) + boh_ref[...]
    h_ref[...] = h

    @pl.when(i == 0)
    def _():
        acc_sum[...] = jnp.zeros_like(acc_sum)
        acc_sq[...] = jnp.zeros_like(acc_sq)

    acc_sum[...] += jnp.sum(h, axis=0, keepdims=True)
    acc_sq[...] += jnp.sum(h * h, axis=0, keepdims=True)

    @pl.when(i == pl.num_programs(0) - 1)
    def _():
        st_ref[0:1, :] = acc_sum[...]
        st_ref[1:2, :] = acc_sq[...]


def _tc_nodes1(acc, op, x, ver2, exp16, woh, boh, dc0, dc1, bn):
    n, d = x.shape
    grid = (n // bn,)
    return pl.pallas_call(
        _nodes1_body,
        grid=grid,
        in_specs=[
            pl.BlockSpec((bn, 640), lambda i: (i, 0)),
            pl.BlockSpec((bn, 128), lambda i: (i, 0)),
            pl.BlockSpec((bn, d), lambda i: (i, 0)),
            pl.BlockSpec((256, 256), lambda i: (0, 0)),
            pl.BlockSpec((16, 256), lambda i: (0, 0)),
            pl.BlockSpec((256, 256), lambda i: (0, 0)),
            pl.BlockSpec((1, 256), lambda i: (0, 0)),
            pl.BlockSpec((1, 256), lambda i: (0, 0)),
            pl.BlockSpec((1, 256), lambda i: (0, 0)),
        ],
        out_specs=[
            pl.BlockSpec((bn, 256), lambda i: (i, 0)),
            pl.BlockSpec((8, 256), lambda i: (0, 0)),
        ],
        out_shape=[
            jax.ShapeDtypeStruct((n, 256), jnp.float32),
            jax.ShapeDtypeStruct((8, 256), jnp.float32),
        ],
        scratch_shapes=[
            pltpu.VMEM((1, 256), jnp.float32),
            pltpu.VMEM((1, 256), jnp.float32),
        ],
    )(acc, op, x, ver2, exp16, woh, boh, dc0, dc1)


def _nodes2_body(h_ref, st_ref, w1_ref, b1_ref, w2_ref, b2_ref, g_ref,
                 bb_ref, cnt_ref, o_ref, st2_ref, acc_sum, acc_sq):
    i = pl.program_id(0)
    cnt = cnt_ref[0, 0]
    mean = st_ref[0:1, :] / cnt
    var = st_ref[1:2, :] / cnt - mean * mean
    hb = g_ref[...] * (h_ref[...] - mean) / jnp.sqrt(var + 1e-5) + bb_ref[...]
    r = jnp.maximum(jnp.dot(hb.astype(jnp.bfloat16), w1_ref[...],
                            preferred_element_type=jnp.float32) + b1_ref[...], 0.0)
    h2 = jnp.dot(r.astype(jnp.bfloat16), w2_ref[...],
                 preferred_element_type=jnp.float32) + b2_ref[...]
    h3 = hb + h2
    o_ref[...] = h3

    @pl.when(i == 0)
    def _():
        acc_sum[...] = jnp.zeros_like(acc_sum)
        acc_sq[...] = jnp.zeros_like(acc_sq)

    acc_sum[...] += jnp.sum(h3, axis=0, keepdims=True)
    acc_sq[...] += jnp.sum(h3 * h3, axis=0, keepdims=True)

    @pl.when(i == pl.num_programs(0) - 1)
    def _():
        st2_ref[0:1, :] = acc_sum[...]
        st2_ref[1:2, :] = acc_sq[...]


def _tc_nodes2(h_pre, stats, w1, b1, w2, b2, g, b, cnt, bn):
    n, d = h_pre.shape
    grid = (n // bn,)
    return pl.pallas_call(
        _nodes2_body,
        grid=grid,
        in_specs=[
            pl.BlockSpec((bn, d), lambda i: (i, 0)),
            pl.BlockSpec((8, 256), lambda i: (0, 0)),
            pl.BlockSpec((256, 512), lambda i: (0, 0)),
            pl.BlockSpec((1, 512), lambda i: (0, 0)),
            pl.BlockSpec((512, 256), lambda i: (0, 0)),
            pl.BlockSpec((1, 256), lambda i: (0, 0)),
            pl.BlockSpec((1, 256), lambda i: (0, 0)),
            pl.BlockSpec((1, 256), lambda i: (0, 0)),
            pl.BlockSpec((1, 1), lambda i: (0, 0), memory_space=pltpu.SMEM),
        ],
        out_specs=[
            pl.BlockSpec((bn, 256), lambda i: (i, 0)),
            pl.BlockSpec((8, 256), lambda i: (0, 0)),
        ],
        out_shape=[
            jax.ShapeDtypeStruct((n, 256), jnp.float32),
            jax.ShapeDtypeStruct((8, 256), jnp.float32),
        ],
        scratch_shapes=[
            pltpu.VMEM((1, 256), jnp.float32),
            pltpu.VMEM((1, 256), jnp.float32),
        ],
    )(h_pre, stats, w1, b1, w2, b2, g, b, cnt)


def _norm_body(v_ref, st_ref, g_ref, b_ref, cnt_ref, o_ref):
    cnt = cnt_ref[0, 0]
    mean = st_ref[0:1, :] / cnt
    var = st_ref[1:2, :] / cnt - mean * mean
    v = v_ref[...].astype(jnp.float32)
    o_ref[...] = (g_ref[...] * (v - mean)
                  / jnp.sqrt(var + 1e-5) + b_ref[...])


def _tc_norm(v, stats, g, b, cnt, bn):
    n, d = v.shape
    grid = (n // bn,)
    return pl.pallas_call(
        _norm_body,
        grid=grid,
        in_specs=[
            pl.BlockSpec((bn, d), lambda i: (i, 0)),
            pl.BlockSpec((8, 256), lambda i: (0, 0)),
            pl.BlockSpec((1, 256), lambda i: (0, 0)),
            pl.BlockSpec((1, 256), lambda i: (0, 0)),
            pl.BlockSpec((1, 1), lambda i: (0, 0), memory_space=pltpu.SMEM),
        ],
        out_specs=pl.BlockSpec((bn, d), lambda i: (i, 0)),
        out_shape=jax.ShapeDtypeStruct((n, d), jnp.float32),
    )(v, stats, g, b, cnt)


def _mesh():
    return plsc.VectorSubcoreMesh(core_axis_name="c", subcore_axis_name="s",
                                  num_cores=NC, num_subcores=NS)


def _sc_gather_kq_v(q, k, v, src3, dst3):
    n = k.shape[0]
    nw = NC * NS
    nch = src3.shape[1]
    e = nw * nch * CH
    epw = nch * CH
    nfull = nch // GB
    tail = nch % GB
    rows_b = GB * CH

    def body(k_hbm, q_hbm, v_hbm, src_hbm, dst_hbm, ks_hbm, qd_hbm, vs_hbm,
             idx_s, idx_d, buf_a, buf_b, sem_a, sem_b):
        wid = lax.axis_index("s") * NC + lax.axis_index("c")
        pltpu.sync_copy(src_hbm.at[wid], idx_s)
        pltpu.sync_copy(dst_hbm.at[wid], idx_d)

        def run_table(tab_hbm, idx_v, out_hbm):
            def fire(bj, buf, sem, cnt):
                ds = []
                for kk in range(cnt):
                    ds.append(pltpu.async_copy(
                        tab_hbm.at[idx_v.at[bj * GB + kk, 0]],
                        buf.at[pl.ds(kk * CH, CH)], sem))
                return ds

            def drain(ds, buf, bj, cnt):
                for cp in ds:
                    cp.wait()
                pltpu.sync_copy(
                    buf.at[pl.ds(0, cnt * CH)],
                    out_hbm.at[pl.ds(wid * epw + bj * rows_b, cnt * CH)])

            def pair(t, carry):
                b0 = t * 2
                b1 = b0 + 1
                da = fire(b0, buf_a, sem_a, GB)
                db = fire(b1, buf_b, sem_b, GB)
                drain(da, buf_a, b0, GB)
                drain(db, buf_b, b1, GB)
                return carry

            lax.fori_loop(0, nfull // 2, pair, 0)
            if nfull % 2:
                bt = nfull - 1
                drain(fire(bt, buf_a, sem_a, GB), buf_a, bt, GB)
            if tail:
                drain(fire(nfull, buf_a, sem_a, tail), buf_a, nfull, tail)

        run_table(k_hbm, idx_s, ks_hbm)
        run_table(q_hbm, idx_d, qd_hbm)
        run_table(v_hbm, idx_s, vs_hbm)

    return pl.kernel(
        body,
        out_type=[
            jax.ShapeDtypeStruct((e, 128), jnp.int32),
            jax.ShapeDtypeStruct((e, 128), jnp.int32),
            jax.ShapeDtypeStruct((e, 128), jnp.int32),
        ],
        mesh=_mesh(),
        scratch_types=[
            pltpu.VMEM((nch, 1, CH), jnp.int32),
            pltpu.VMEM((nch, 1, CH), jnp.int32),
            pltpu.VMEM((rows_b, 128), jnp.int32),
            pltpu.VMEM((rows_b, 128), jnp.int32),
            pltpu.SemaphoreType.DMA,
            pltpu.SemaphoreType.DMA,
        ],
    )(k, q, v, src3, dst3)


def _sc_scatter640(payload, dst3, dst3b, zeros128):
    n = zeros128.shape[0]
    nchb = dst3b.shape[1]
    nch = dst3.shape[1]
    eps = nchb * CHB
    epw = nch * CH
    rpw = n // NS

    def body(pay_hbm, dst_hbm, dstb_hbm, z_hbm, out_hbm, outp_hbm,
             idxb, idx4, pbuf_a, pbuf_b, accsh, sem_a, sem_b):
        c = lax.axis_index("c")
        s = lax.axis_index("s")
        rbase = s * rpw
        wid = c * NS + s
        pltpu.sync_copy(dstb_hbm.at[s], idxb)

        def run(col, idx_row, ch, nblk, ebase):
            def fire(bj, buf, sem):
                return pltpu.async_copy(
                    pay_hbm.at[pl.ds(ebase + bj * ch, ch), pl.ds(col, 128)],
                    buf.at[pl.ds(0, ch)], sem)

            def scat(bj, buf):
                pltpu.sync_copy(buf.at[pl.ds(0, ch)],
                                accsh.at[idx_row(bj)], add=True)

            def pair(t, carry):
                b0 = t * 2
                b1 = b0 + 1
                da = fire(b0, pbuf_a, sem_a)
                db = fire(b1, pbuf_b, sem_b)
                da.wait()
                scat(b0, pbuf_a)
                db.wait()
                scat(b1, pbuf_b)
                return carry

            lax.fori_loop(0, nblk // 2, pair, 0)
            if nblk % 2:
                bt = nblk - 1
                d = fire(bt, pbuf_a, sem_a)
                d.wait()
                scat(bt, pbuf_a)

        for f in range(2):
            col = (c * 2 + f) * 128
            pltpu.sync_copy(z_hbm.at[pl.ds(rbase, rpw)],
                            accsh.at[pl.ds(rbase, rpw)])
            plsc.subcore_barrier()
            run(col, lambda bj: idxb.at[bj, 0], CHB, nchb, s * eps)
            plsc.subcore_barrier()
            pltpu.sync_copy(accsh.at[pl.ds(rbase, rpw)],
                            out_hbm.at[pl.ds(rbase, rpw), pl.ds(col, 128)])
            plsc.subcore_barrier()

        pltpu.sync_copy(z_hbm.at[pl.ds(rbase, rpw)],
                        accsh.at[pl.ds(rbase, rpw)])
        plsc.subcore_barrier()
        def idx4_row(bj):
            pltpu.sync_copy(dst_hbm.at[wid, bj, 0], idx4)
            return idx4

        run(512, idx4_row, CH, nch, wid * epw)
        plsc.subcore_barrier()

        @pl.when(c == 0)
        def _():
            pltpu.sync_copy(accsh.at[pl.ds(rbase, rpw)],
                            out_hbm.at[pl.ds(rbase, rpw), pl.ds(512, 128)])

        @pl.when(c == 1)
        def _():
            pltpu.sync_copy(accsh.at[pl.ds(rbase, rpw)],
                            outp_hbm.at[pl.ds(rbase, rpw)])

    return pl.kernel(
        body,
        out_type=[
            jax.ShapeDtypeStruct((n, 640), jnp.float32),
            jax.ShapeDtypeStruct((n, 128), jnp.float32),
        ],
        mesh=_mesh(),
        scratch_types=[
            pltpu.VMEM((nchb, 1, CHB), jnp.int32),
            pltpu.VMEM((CH,), jnp.int32),
            pltpu.VMEM((CHB, 128), jnp.float32),
            pltpu.VMEM((CHB, 128), jnp.float32),
            pltpu.VMEM_SHARED((n, 128), jnp.float32),
            pltpu.SemaphoreType.DMA,
            pltpu.SemaphoreType.DMA,
        ],
    )(payload, dst3, dst3b, zeros128)


def kernel(x, edge_index, edge_attr, params):
    n, d = x.shape
    e = edge_attr.shape[0]
    src = edge_index[0]
    dst = edge_index[1]

    bn_blk = 1000 if n % 1000 == 0 else n // 10
    be_blk = 640 if e % 640 == 0 else e // 10

    wqkv = jnp.concatenate([params['Wq'], params['Wk'], params['Wv']], axis=1)
    bqkv = jnp.concatenate(
        [params['bq'], jnp.zeros((512,), jnp.float32)])[None, :]
    cw = np.arange(H)[:, None] * 64 + np.arange(DH)[None, :]
    cb = cw + DH
    we_w = params['We'][:, cw.reshape(-1)]
    we_b = params['We'][:, cb.reshape(-1)]
    be_w = params['be'][cw.reshape(-1)][None, :]
    be_b = params['be'][cb.reshape(-1)][None, :]
    awf = params['Aw'][:, :, 0].T.reshape(-1)
    eye16 = np.kron(np.eye(H, 16, dtype=np.float32), np.ones((DH, 1), np.float32))
    aw2 = awf[:, None] * eye16
    exp16_np = np.zeros((16, 256), np.float32)
    for hh in range(H):
        exp16_np[hh, hh * DH:(hh + 1) * DH] = 1.0
    exp16 = jnp.asarray(exp16_np)
    ver = params['VeRow']
    ver2 = jnp.zeros((256, 256), jnp.float32)
    for hh in range(H):
        ver2 = ver2.at[hh * DH:(hh + 1) * DH, hh * DH:(hh + 1) * DH].set(ver[:, hh, :])
    dc0 = params['deg_coef'][0, :, 0][None, :]
    dc1 = params['deg_coef'][0, :, 1][None, :]
    boh = params['boh'][None, :]
    boe = params['boe'][None, :]
    bf1 = params['bf1'][None, :]
    bf2 = params['bf2'][None, :]
    g1h = params['g1h'][None, :]
    b1h = params['b1h'][None, :]
    g1e = params['g1e'][None, :]
    b1e = params['b1e'][None, :]
    g2h = params['g2h'][None, :]
    b2h = params['b2h'][None, :]
    cnt_n = jnp.full((1, 1), float(n), jnp.float32)
    cnt_e = jnp.full((1, 1), float(e), jnp.float32)

    nw = NC * NS
    nch = e // (nw * CH)
    src3 = src.reshape(nw, nch, 1, CH)
    dst3 = dst.reshape(nw, nch, 1, CH)
    dst3b = dst.reshape(NS, e // (NS * CHB), 1, CHB)
    npad = ((n + 8 * NS - 1) // (8 * NS)) * (8 * NS)
    zeros128 = jnp.zeros((npad, 128), jnp.float32)

    bf = jnp.bfloat16
    wqkv = wqkv.astype(bf)
    we_w = we_w.astype(bf)
    we_b = we_b.astype(bf)
    aw2 = jnp.asarray(aw2).astype(bf)
    ver2 = ver2.astype(bf)
    woe_bf = params['Woe'].astype(bf)
    woh_bf = params['Woh'].astype(bf)
    w1_bf = params['W1'].astype(bf)
    w2_bf = params['W2'].astype(bf)

    q, k, v = _tc_nodeproj(x, wqkv, bqkv, bn_blk)
    def to_i32(t):
        return lax.bitcast_convert_type(t.reshape(n, 128, 2), jnp.int32)

    def to_bf16(t):
        return lax.bitcast_convert_type(t, jnp.bfloat16).reshape(e, 256)

    ks3, qd3, vs3 = _sc_gather_kq_v(to_i32(q), to_i32(k), to_i32(v),
                                    src3, dst3)
    ks = to_bf16(ks3)
    qd = to_bf16(qd3)
    vs = to_bf16(vs3)
    payload, ee_pre, ee_stats = _tc_edges(
        ks, qd, edge_attr, vs, we_w, we_b, be_w, be_b, aw2, exp16,
        woe_bf, boe, be_blk)
    acc, accp = _sc_scatter640(payload, dst3, dst3b, zeros128)
    h_pre, h_stats = _tc_nodes1(acc, accp, x, ver2, exp16, woh_bf,
                                boh, dc0, dc1, bn_blk)
    h3, h_stats2 = _tc_nodes2(h_pre, h_stats, w1_bf, bf1,
                              w2_bf, bf2, g1h, b1h, cnt_n, bn_blk)
    h_out = _tc_norm(h3, h_stats2, g2h, b2h, cnt_n, bn_blk)
    ee_out = _tc_norm(ee_pre, ee_stats, g1e, b1e, cnt_e, be_blk)
    return h_out, ee_out


# speedup vs baseline: 2.3851x; 2.3851x over previous
import jax
import jax.numpy as jnp
import numpy as np
from jax import lax
from jax.experimental import pallas as pl
from jax.experimental.pallas import tpu as pltpu
from jax.experimental.pallas import tpu_sc as plsc

H = 8
DH = 32
NC = 2
NS = 16
CH = 40
CHB = 80
GB = 2


def _nodeproj_body(x_ref, w_ref, b_ref, q_ref, k_ref, v_ref):
    y = jnp.dot(x_ref[...].astype(jnp.bfloat16), w_ref[...],
                preferred_element_type=jnp.float32)
    y = y + b_ref[...]
    q_ref[...] = y[:, 0:256]
    k_ref[...] = y[:, 256:512]
    v_ref[...] = y[:, 512:768]


def _tc_nodeproj(x, wqkv, bqkv, bn):
    n, d = x.shape
    grid = (n // bn,)
    return pl.pallas_call(
        _nodeproj_body,
        grid=grid,
        in_specs=[
            pl.BlockSpec((bn, d), lambda i: (i, 0)),
            pl.BlockSpec((d, 768), lambda i: (0, 0)),
            pl.BlockSpec((1, 768), lambda i: (0, 0)),
        ],
        out_specs=[
            pl.BlockSpec((bn, 256), lambda i: (i, 0)),
            pl.BlockSpec((bn, 256), lambda i: (i, 0)),
            pl.BlockSpec((bn, 256), lambda i: (i, 0)),
        ],
        out_shape=[
            jax.ShapeDtypeStruct((n, 256), jnp.float32),
            jax.ShapeDtypeStruct((n, 256), jnp.float32),
            jax.ShapeDtypeStruct((n, 256), jnp.float32),
        ],
    )(x, wqkv, bqkv)


def _edges_body(ks_ref, qd_ref, ea_ref, vs_ref, ww_ref, wb_ref, bw_ref,
                bb_ref, aw_ref, exp_ref, woe_ref, boe_ref,
                pay_ref, ee_ref, st_ref, acc_sum, acc_sq):
    i = pl.program_id(0)
    ea = ea_ref[...]
    ea_bf = ea.astype(jnp.bfloat16)
    ew = jnp.dot(ea_bf, ww_ref[...], preferred_element_type=jnp.float32) + bw_ref[...]
    eb = jnp.dot(ea_bf, wb_ref[...], preferred_element_type=jnp.float32) + bb_ref[...]
    s = (ks_ref[...].astype(jnp.float32)
         + qd_ref[...].astype(jnp.float32)) * ew
    s = jnp.sign(s) * jnp.sqrt(jnp.abs(s))
    e_t = jnp.maximum(s + eb, 0.0)
    e_bf = e_t.astype(jnp.bfloat16)
    att = jnp.dot(e_bf, aw_ref[...], preferred_element_type=jnp.float32)
    att = jnp.clip(att, -5.0, 5.0)
    xa = jnp.exp(att)
    p256 = jnp.dot(xa, exp_ref[...], preferred_element_type=jnp.float32)
    pay_ref[:, 0:256] = vs_ref[...].astype(jnp.float32) * p256
    pay_ref[:, 256:512] = e_t * p256
    pay_ref[:, 512:528] = xa
    pay_ref[:, 528:640] = jnp.zeros((ea.shape[0], 112), jnp.float32)
    ee = ea + jnp.dot(e_bf, woe_ref[...],
                      preferred_element_type=jnp.float32) + boe_ref[...]
    ee_ref[...] = ee.astype(jnp.bfloat16)

    @pl.when(i == 0)
    def _():
        acc_sum[...] = jnp.zeros_like(acc_sum)
        acc_sq[...] = jnp.zeros_like(acc_sq)

    acc_sum[...] += jnp.sum(ee, axis=0, keepdims=True)
    acc_sq[...] += jnp.sum(ee * ee, axis=0, keepdims=True)

    @pl.when(i == pl.num_programs(0) - 1)
    def _():
        st_ref[0:1, :] = acc_sum[...]
        st_ref[1:2, :] = acc_sq[...]


def _tc_edges(ks, qd, edge_attr, vs, we_w, we_b, be_w, be_b, aw2, exp16,
              woe, boe, be_blk):
    e, d = edge_attr.shape
    grid = (e // be_blk,)
    return pl.pallas_call(
        _edges_body,
        grid=grid,
        in_specs=[
            pl.BlockSpec((be_blk, 256), lambda i: (i, 0)),
            pl.BlockSpec((be_blk, 256), lambda i: (i, 0)),
            pl.BlockSpec((be_blk, d), lambda i: (i, 0)),
            pl.BlockSpec((be_blk, 256), lambda i: (i, 0)),
            pl.BlockSpec((d, 256), lambda i: (0, 0)),
            pl.BlockSpec((d, 256), lambda i: (0, 0)),
            pl.BlockSpec((1, 256), lambda i: (0, 0)),
            pl.BlockSpec((1, 256), lambda i: (0, 0)),
            pl.BlockSpec((256, 16), lambda i: (0, 0)),
            pl.BlockSpec((16, 256), lambda i: (0, 0)),
            pl.BlockSpec((256, 256), lambda i: (0, 0)),
            pl.BlockSpec((1, 256), lambda i: (0, 0)),
        ],
        out_specs=[
            pl.BlockSpec((be_blk, 640), lambda i: (i, 0)),
            pl.BlockSpec((be_blk, 256), lambda i: (i, 0)),
            pl.BlockSpec((8, 256), lambda i: (0, 0)),
        ],
        out_shape=[
            jax.ShapeDtypeStruct((e, 640), jnp.float32),
            jax.ShapeDtypeStruct((e, 256), jnp.bfloat16),
            jax.ShapeDtypeStruct((8, 256), jnp.float32),
        ],
        scratch_shapes=[
            pltpu.VMEM((1, 256), jnp.float32),
            pltpu.VMEM((1, 256), jnp.float32),
        ],
    )(ks, qd, edge_attr, vs, we_w, we_b, be_w, be_b, aw2, exp16, woe, boe)


def _nodes1_body(acc_ref, op_ref, x_ref, ver_ref, exp_ref, woh_ref, boh_ref,
                 dc0_ref, dc1_ref, h_ref, st_ref, acc_sum, acc_sq):
    i = pl.program_id(0)
    s16 = acc_ref[:, 512:528] + op_ref[:, 0:16]
    rs16 = 1.0 / (s16 + 1e-30)
    rs256 = jnp.dot(rs16, exp_ref[...], preferred_element_type=jnp.float32)
    ld = jnp.log(s16[:, 8:9] + 1.0)
    wv = (acc_ref[:, 0:256]
          + jnp.dot(acc_ref[:, 256:512].astype(jnp.bfloat16), ver_ref[...],
                    preferred_element_type=jnp.float32))
    wv = wv * rs256
    hh = wv * dc0_ref[...] + (wv * ld) * dc1_ref[...]
    h = x_ref[...] + jnp.dot(hh.astype(jnp.bfloat16), woh_ref[...],
                             preferred_element_type=jnp.float32) + boh_ref[...]
    h_ref[...] = h

    @pl.when(i == 0)
    def _():
        acc_sum[...] = jnp.zeros_like(acc_sum)
        acc_sq[...] = jnp.zeros_like(acc_sq)

    acc_sum[...] += jnp.sum(h, axis=0, keepdims=True)
    acc_sq[...] += jnp.sum(h * h, axis=0, keepdims=True)

    @pl.when(i == pl.num_programs(0) - 1)
    def _():
        st_ref[0:1, :] = acc_sum[...]
        st_ref[1:2, :] = acc_sq[...]


def _tc_nodes1(acc, op, x, ver2, exp16, woh, boh, dc0, dc1, bn):
    n, d = x.shape
    grid = (n // bn,)
    return pl.pallas_call(
        _nodes1_body,
        grid=grid,
        in_specs=[
            pl.BlockSpec((bn, 640), lambda i: (i, 0)),
            pl.BlockSpec((bn, 128), lambda i: (i, 0)),
            pl.BlockSpec((bn, d), lambda i: (i, 0)),
            pl.BlockSpec((256, 256), lambda i: (0, 0)),
            pl.BlockSpec((16, 256), lambda i: (0, 0)),
            pl.BlockSpec((256, 256), lambda i: (0, 0)),
            pl.BlockSpec((1, 256), lambda i: (0, 0)),
            pl.BlockSpec((1, 256), lambda i: (0, 0)),
            pl.BlockSpec((1, 256), lambda i: (0, 0)),
        ],
        out_specs=[
            pl.BlockSpec((bn, 256), lambda i: (i, 0)),
            pl.BlockSpec((8, 256), lambda i: (0, 0)),
        ],
        out_shape=[
            jax.ShapeDtypeStruct((n, 256), jnp.float32),
            jax.ShapeDtypeStruct((8, 256), jnp.float32),
        ],
        scratch_shapes=[
            pltpu.VMEM((1, 256), jnp.float32),
            pltpu.VMEM((1, 256), jnp.float32),
        ],
    )(acc, op, x, ver2, exp16, woh, boh, dc0, dc1)


def _nodes2_body(h_ref, st_ref, w1_ref, b1_ref, w2_ref, b2_ref, g_ref,
                 bb_ref, cnt_ref, o_ref, st2_ref, acc_sum, acc_sq):
    i = pl.program_id(0)
    cnt = cnt_ref[0, 0]
    mean = st_ref[0:1, :] / cnt
    var = st_ref[1:2, :] / cnt - mean * mean
    hb = g_ref[...] * (h_ref[...] - mean) / jnp.sqrt(var + 1e-5) + bb_ref[...]
    r = jnp.maximum(jnp.dot(hb.astype(jnp.bfloat16), w1_ref[...],
                            preferred_element_type=jnp.float32) + b1_ref[...], 0.0)
    h2 = jnp.dot(r.astype(jnp.bfloat16), w2_ref[...],
                 preferred_element_type=jnp.float32) + b2_ref[...]
    h3 = hb + h2
    o_ref[...] = h3

    @pl.when(i == 0)
    def _():
        acc_sum[...] = jnp.zeros_like(acc_sum)
        acc_sq[...] = jnp.zeros_like(acc_sq)

    acc_sum[...] += jnp.sum(h3, axis=0, keepdims=True)
    acc_sq[...] += jnp.sum(h3 * h3, axis=0, keepdims=True)

    @pl.when(i == pl.num_programs(0) - 1)
    def _():
        st2_ref[0:1, :] = acc_sum[...]
        st2_ref[1:2, :] = acc_sq[...]


def _tc_nodes2(h_pre, stats, w1, b1, w2, b2, g, b, cnt, bn):
    n, d = h_pre.shape
    grid = (n // bn,)
    return pl.pallas_call(
        _nodes2_body,
        grid=grid,
        in_specs=[
            pl.BlockSpec((bn, d), lambda i: (i, 0)),
            pl.BlockSpec((8, 256), lambda i: (0, 0)),
            pl.BlockSpec((256, 512), lambda i: (0, 0)),
            pl.BlockSpec((1, 512), lambda i: (0, 0)),
            pl.BlockSpec((512, 256), lambda i: (0, 0)),
            pl.BlockSpec((1, 256), lambda i: (0, 0)),
            pl.BlockSpec((1, 256), lambda i: (0, 0)),
            pl.BlockSpec((1, 256), lambda i: (0, 0)),
            pl.BlockSpec((1, 1), lambda i: (0, 0), memory_space=pltpu.SMEM),
        ],
        out_specs=[
            pl.BlockSpec((bn, 256), lambda i: (i, 0)),
            pl.BlockSpec((8, 256), lambda i: (0, 0)),
        ],
        out_shape=[
            jax.ShapeDtypeStruct((n, 256), jnp.float32),
            jax.ShapeDtypeStruct((8, 256), jnp.float32),
        ],
        scratch_shapes=[
            pltpu.VMEM((1, 256), jnp.float32),
            pltpu.VMEM((1, 256), jnp.float32),
        ],
    )(h_pre, stats, w1, b1, w2, b2, g, b, cnt)


def _norm_body(v_ref, st_ref, g_ref, b_ref, cnt_ref, o_ref):
    cnt = cnt_ref[0, 0]
    mean = st_ref[0:1, :] / cnt
    var = st_ref[1:2, :] / cnt - mean * mean
    v = v_ref[...].astype(jnp.float32)
    o_ref[...] = (g_ref[...] * (v - mean)
                  / jnp.sqrt(var + 1e-5) + b_ref[...])


def _tc_norm(v, stats, g, b, cnt, bn):
    n, d = v.shape
    grid = (n // bn,)
    return pl.pallas_call(
        _norm_body,
        grid=grid,
        in_specs=[
            pl.BlockSpec((bn, d), lambda i: (i, 0)),
            pl.BlockSpec((8, 256), lambda i: (0, 0)),
            pl.BlockSpec((1, 256), lambda i: (0, 0)),
            pl.BlockSpec((1, 256), lambda i: (0, 0)),
            pl.BlockSpec((1, 1), lambda i: (0, 0), memory_space=pltpu.SMEM),
        ],
        out_specs=pl.BlockSpec((bn, d), lambda i: (i, 0)),
        out_shape=jax.ShapeDtypeStruct((n, d), jnp.float32),
    )(v, stats, g, b, cnt)


def _mesh():
    return plsc.VectorSubcoreMesh(core_axis_name="c", subcore_axis_name="s",
                                  num_cores=NC, num_subcores=NS)


def _sc_gather_kq_v(q, k, v, src3, dst3):
    n, d = k.shape
    nw = NC * NS
    nch = src3.shape[1]
    e = nw * nch * CH
    epw = nch * CH
    nfull = nch // GB
    tail = nch % GB
    rows_b = GB * CH

    def body(k_hbm, q_hbm, v_hbm, src_hbm, dst_hbm, ks_hbm, qd_hbm, vs_hbm,
             idx_s, idx_d, buf_a, buf_b, sem_a, sem_b):
        wid = lax.axis_index("s") * NC + lax.axis_index("c")
        pltpu.sync_copy(src_hbm.at[wid], idx_s)
        pltpu.sync_copy(dst_hbm.at[wid], idx_d)

        def run_table(tab_hbm, idx_v, out_hbm):
            def fire(bj, buf, sem, cnt):
                ds = []
                for kk in range(cnt):
                    ds.append(pltpu.async_copy(
                        tab_hbm.at[idx_v.at[bj * GB + kk, 0]],
                        buf.at[pl.ds(kk * CH, CH)], sem))
                return ds

            def drain(ds, buf, bj, cnt):
                for cp in ds:
                    cp.wait()
                pltpu.sync_copy(
                    buf.at[pl.ds(0, cnt * CH)],
                    out_hbm.at[pl.ds(wid * epw + bj * rows_b, cnt * CH)])

            def pair(t, carry):
                b0 = t * 2
                b1 = b0 + 1
                da = fire(b0, buf_a, sem_a, GB)
                db = fire(b1, buf_b, sem_b, GB)
                drain(da, buf_a, b0, GB)
                drain(db, buf_b, b1, GB)
                return carry

            lax.fori_loop(0, nfull // 2, pair, 0)
            if nfull % 2:
                bt = nfull - 1
                drain(fire(bt, buf_a, sem_a, GB), buf_a, bt, GB)
            if tail:
                drain(fire(nfull, buf_a, sem_a, tail), buf_a, nfull, tail)

        run_table(k_hbm, idx_s, ks_hbm)
        run_table(q_hbm, idx_d, qd_hbm)
        run_table(v_hbm, idx_s, vs_hbm)

    return pl.kernel(
        body,
        out_type=[
            jax.ShapeDtypeStruct((e, d), jnp.float32),
            jax.ShapeDtypeStruct((e, d), jnp.float32),
            jax.ShapeDtypeStruct((e, d), jnp.float32),
        ],
        mesh=_mesh(),
        scratch_types=[
            pltpu.VMEM((nch, 1, CH), jnp.int32),
            pltpu.VMEM((nch, 1, CH), jnp.int32),
            pltpu.VMEM((rows_b, d), jnp.float32),
            pltpu.VMEM((rows_b, d), jnp.float32),
            pltpu.SemaphoreType.DMA,
            pltpu.SemaphoreType.DMA,
        ],
    )(k, q, v, src3, dst3)


def _sc_scatter640(payload, dst3, dst3b, zeros128):
    n = zeros128.shape[0]
    nchb = dst3b.shape[1]
    nch = dst3.shape[1]
    eps = nchb * CHB
    epw = nch * CH
    rpw = n // NS

    def body(pay_hbm, dst_hbm, dstb_hbm, z_hbm, out_hbm, outp_hbm,
             idxb, idx4, pbuf_a, pbuf_b, accsh, sem_a, sem_b):
        c = lax.axis_index("c")
        s = lax.axis_index("s")
        rbase = s * rpw
        wid = c * NS + s
        pltpu.sync_copy(dstb_hbm.at[s], idxb)

        def run(col, idx_row, ch, nblk, ebase):
            def fire(bj, buf, sem):
                return pltpu.async_copy(
                    pay_hbm.at[pl.ds(ebase + bj * ch, ch), pl.ds(col, 128)],
                    buf.at[pl.ds(0, ch)], sem)

            def scat(bj, buf):
                pltpu.sync_copy(buf.at[pl.ds(0, ch)],
                                accsh.at[idx_row(bj)], add=True)

            def pair(t, carry):
                b0 = t * 2
                b1 = b0 + 1
                da = fire(b0, pbuf_a, sem_a)
                db = fire(b1, pbuf_b, sem_b)
                da.wait()
                scat(b0, pbuf_a)
                db.wait()
                scat(b1, pbuf_b)
                return carry

            lax.fori_loop(0, nblk // 2, pair, 0)
            if nblk % 2:
                bt = nblk - 1
                d = fire(bt, pbuf_a, sem_a)
                d.wait()
                scat(bt, pbuf_a)

        for f in range(2):
            col = (c * 2 + f) * 128
            pltpu.sync_copy(z_hbm.at[pl.ds(rbase, rpw)],
                            accsh.at[pl.ds(rbase, rpw)])
            plsc.subcore_barrier()
            run(col, lambda bj: idxb.at[bj, 0], CHB, nchb, s * eps)
            plsc.subcore_barrier()
            pltpu.sync_copy(accsh.at[pl.ds(rbase, rpw)],
                            out_hbm.at[pl.ds(rbase, rpw), pl.ds(col, 128)])
            plsc.subcore_barrier()

        pltpu.sync_copy(z_hbm.at[pl.ds(rbase, rpw)],
                        accsh.at[pl.ds(rbase, rpw)])
        plsc.subcore_barrier()
        def idx4_row(bj):
            pltpu.sync_copy(dst_hbm.at[wid, bj, 0], idx4)
            return idx4

        run(512, idx4_row, CH, nch, wid * epw)
        plsc.subcore_barrier()

        @pl.when(c == 0)
        def _():
            pltpu.sync_copy(accsh.at[pl.ds(rbase, rpw)],
                            out_hbm.at[pl.ds(rbase, rpw), pl.ds(512, 128)])

        @pl.when(c == 1)
        def _():
            pltpu.sync_copy(accsh.at[pl.ds(rbase, rpw)],
                            outp_hbm.at[pl.ds(rbase, rpw)])

    return pl.kernel(
        body,
        out_type=[
            jax.ShapeDtypeStruct((n, 640), jnp.float32),
            jax.ShapeDtypeStruct((n, 128), jnp.float32),
        ],
        mesh=_mesh(),
        scratch_types=[
            pltpu.VMEM((nchb, 1, CHB), jnp.int32),
            pltpu.VMEM((CH,), jnp.int32),
            pltpu.VMEM((CHB, 128), jnp.float32),
            pltpu.VMEM((CHB, 128), jnp.float32),
            pltpu.VMEM_SHARED((n, 128), jnp.float32),
            pltpu.SemaphoreType.DMA,
            pltpu.SemaphoreType.DMA,
        ],
    )(payload, dst3, dst3b, zeros128)


def kernel(x, edge_index, edge_attr, params):
    n, d = x.shape
    e = edge_attr.shape[0]
    src = edge_index[0]
    dst = edge_index[1]

    bn_blk = 1000 if n % 1000 == 0 else n // 10
    be_blk = 640 if e % 640 == 0 else e // 10

    wqkv = jnp.concatenate([params['Wq'], params['Wk'], params['Wv']], axis=1)
    bqkv = jnp.concatenate(
        [params['bq'], jnp.zeros((512,), jnp.float32)])[None, :]
    cw = np.arange(H)[:, None] * 64 + np.arange(DH)[None, :]
    cb = cw + DH
    we_w = params['We'][:, cw.reshape(-1)]
    we_b = params['We'][:, cb.reshape(-1)]
    be_w = params['be'][cw.reshape(-1)][None, :]
    be_b = params['be'][cb.reshape(-1)][None, :]
    awf = params['Aw'][:, :, 0].T.reshape(-1)
    eye16 = np.kron(np.eye(H, 16, dtype=np.float32), np.ones((DH, 1), np.float32))
    aw2 = awf[:, None] * eye16
    exp16_np = np.zeros((16, 256), np.float32)
    for hh in range(H):
        exp16_np[hh, hh * DH:(hh + 1) * DH] = 1.0
    exp16 = jnp.asarray(exp16_np)
    ver = params['VeRow']
    ver2 = jnp.zeros((256, 256), jnp.float32)
    for hh in range(H):
        ver2 = ver2.at[hh * DH:(hh + 1) * DH, hh * DH:(hh + 1) * DH].set(ver[:, hh, :])
    dc0 = params['deg_coef'][0, :, 0][None, :]
    dc1 = params['deg_coef'][0, :, 1][None, :]
    boh = params['boh'][None, :]
    boe = params['boe'][None, :]
    bf1 = params['bf1'][None, :]
    bf2 = params['bf2'][None, :]
    g1h = params['g1h'][None, :]
    b1h = params['b1h'][None, :]
    g1e = params['g1e'][None, :]
    b1e = params['b1e'][None, :]
    g2h = params['g2h'][None, :]
    b2h = params['b2h'][None, :]
    cnt_n = jnp.full((1, 1), float(n), jnp.float32)
    cnt_e = jnp.full((1, 1), float(e), jnp.float32)

    nw = NC * NS
    nch = e // (nw * CH)
    src3 = src.reshape(nw, nch, 1, CH)
    dst3 = dst.reshape(nw, nch, 1, CH)
    dst3b = dst.reshape(NS, e // (NS * CHB), 1, CHB)
    npad = ((n + 8 * NS - 1) // (8 * NS)) * (8 * NS)
    zeros128 = jnp.zeros((npad, 128), jnp.float32)

    bf = jnp.bfloat16
    wqkv = wqkv.astype(bf)
    we_w = we_w.astype(bf)
    we_b = we_b.astype(bf)
    aw2 = jnp.asarray(aw2).astype(bf)
    ver2 = ver2.astype(bf)
    woe_bf = params['Woe'].astype(bf)
    woh_bf = params['Woh'].astype(bf)
    w1_bf = params['W1'].astype(bf)
    w2_bf = params['W2'].astype(bf)

    q, k, v = _tc_nodeproj(x, wqkv, bqkv, bn_blk)
    ks, qd, vs = _sc_gather_kq_v(q, k, v, src3, dst3)
    payload, ee_pre, ee_stats = _tc_edges(
        ks, qd, edge_attr, vs, we_w, we_b, be_w, be_b, aw2, exp16,
        woe_bf, boe, be_blk)
    acc, accp = _sc_scatter640(payload, dst3, dst3b, zeros128)
    h_pre, h_stats = _tc_nodes1(acc, accp, x, ver2, exp16, woh_bf,
                                boh, dc0, dc1, bn_blk)
    h3, h_stats2 = _tc_nodes2(h_pre, h_stats, w1_bf, bf1,
                              w2_bf, bf2, g1h, b1h, cnt_n, bn_blk)
    h_out = _tc_norm(h3, h_stats2, g2h, b2h, cnt_n, bn_blk)
    ee_out = _tc_norm(ee_pre, ee_stats, g1e, b1e, cnt_e, be_blk)
    return h_out, ee_out


# speedup vs baseline: 2.5913x; 1.0864x over previous
import jax
import jax.numpy as jnp
import numpy as np
from jax import lax
from jax.experimental import pallas as pl
from jax.experimental.pallas import tpu as pltpu
from jax.experimental.pallas import tpu_sc as plsc

H = 8
DH = 32
NC = 2
NS = 16
CH = 40
CHB = 80
GB = 4


def _nodeproj_body(x_ref, w_ref, b_ref, q_ref, k_ref, v_ref):
    y = jnp.dot(x_ref[...].astype(jnp.bfloat16), w_ref[...],
                preferred_element_type=jnp.float32)
    y = y + b_ref[...]
    q_ref[...] = y[:, 0:256]
    k_ref[...] = y[:, 256:512]
    v_ref[...] = y[:, 512:768]


def _tc_nodeproj(x, wqkv, bqkv, bn):
    n, d = x.shape
    grid = (n // bn,)
    return pl.pallas_call(
        _nodeproj_body,
        grid=grid,
        in_specs=[
            pl.BlockSpec((bn, d), lambda i: (i, 0)),
            pl.BlockSpec((d, 768), lambda i: (0, 0)),
            pl.BlockSpec((1, 768), lambda i: (0, 0)),
        ],
        out_specs=[
            pl.BlockSpec((bn, 256), lambda i: (i, 0)),
            pl.BlockSpec((bn, 256), lambda i: (i, 0)),
            pl.BlockSpec((bn, 256), lambda i: (i, 0)),
        ],
        out_shape=[
            jax.ShapeDtypeStruct((n, 256), jnp.float32),
            jax.ShapeDtypeStruct((n, 256), jnp.float32),
            jax.ShapeDtypeStruct((n, 256), jnp.float32),
        ],
    )(x, wqkv, bqkv)


def _edges_body(ks_ref, qd_ref, ea_ref, vs_ref, ww_ref, wb_ref, bw_ref,
                bb_ref, aw_ref, exp_ref, woe_ref, boe_ref,
                pay_ref, ee_ref, st_ref, acc_sum, acc_sq):
    i = pl.program_id(0)
    ea = ea_ref[...]
    ea_bf = ea.astype(jnp.bfloat16)
    ew = jnp.dot(ea_bf, ww_ref[...], preferred_element_type=jnp.float32) + bw_ref[...]
    eb = jnp.dot(ea_bf, wb_ref[...], preferred_element_type=jnp.float32) + bb_ref[...]
    s = (ks_ref[...].astype(jnp.float32)
         + qd_ref[...].astype(jnp.float32)) * ew
    s = jnp.sign(s) * jnp.sqrt(jnp.abs(s))
    e_t = jnp.maximum(s + eb, 0.0)
    e_bf = e_t.astype(jnp.bfloat16)
    att = jnp.dot(e_bf, aw_ref[...], preferred_element_type=jnp.float32)
    att = jnp.clip(att, -5.0, 5.0)
    xa = jnp.exp(att)
    p256 = jnp.dot(xa, exp_ref[...], preferred_element_type=jnp.float32)
    pay_ref[:, 0:256] = vs_ref[...].astype(jnp.float32) * p256
    pay_ref[:, 256:512] = e_t * p256
    pay_ref[:, 512:528] = xa
    pay_ref[:, 528:640] = jnp.zeros((ea.shape[0], 112), jnp.float32)
    ee = ea + jnp.dot(e_bf, woe_ref[...],
                      preferred_element_type=jnp.float32) + boe_ref[...]
    ee_ref[...] = ee.astype(jnp.bfloat16)

    @pl.when(i == 0)
    def _():
        acc_sum[...] = jnp.zeros_like(acc_sum)
        acc_sq[...] = jnp.zeros_like(acc_sq)

    acc_sum[...] += jnp.sum(ee, axis=0, keepdims=True)
    acc_sq[...] += jnp.sum(ee * ee, axis=0, keepdims=True)

    @pl.when(i == pl.num_programs(0) - 1)
    def _():
        st_ref[0:1, :] = acc_sum[...]
        st_ref[1:2, :] = acc_sq[...]


def _tc_edges(ks, qd, edge_attr, vs, we_w, we_b, be_w, be_b, aw2, exp16,
              woe, boe, be_blk):
    e, d = edge_attr.shape
    grid = (e // be_blk,)
    return pl.pallas_call(
        _edges_body,
        grid=grid,
        in_specs=[
            pl.BlockSpec((be_blk, 256), lambda i: (i, 0)),
            pl.BlockSpec((be_blk, 256), lambda i: (i, 0)),
            pl.BlockSpec((be_blk, d), lambda i: (i, 0)),
            pl.BlockSpec((be_blk, 256), lambda i: (i, 0)),
            pl.BlockSpec((d, 256), lambda i: (0, 0)),
            pl.BlockSpec((d, 256), lambda i: (0, 0)),
            pl.BlockSpec((1, 256), lambda i: (0, 0)),
            pl.BlockSpec((1, 256), lambda i: (0, 0)),
            pl.BlockSpec((256, 16), lambda i: (0, 0)),
            pl.BlockSpec((16, 256), lambda i: (0, 0)),
            pl.BlockSpec((256, 256), lambda i: (0, 0)),
            pl.BlockSpec((1, 256), lambda i: (0, 0)),
        ],
        out_specs=[
            pl.BlockSpec((be_blk, 640), lambda i: (i, 0)),
            pl.BlockSpec((be_blk, 256), lambda i: (i, 0)),
            pl.BlockSpec((8, 256), lambda i: (0, 0)),
        ],
        out_shape=[
            jax.ShapeDtypeStruct((e, 640), jnp.float32),
            jax.ShapeDtypeStruct((e, 256), jnp.bfloat16),
            jax.ShapeDtypeStruct((8, 256), jnp.float32),
        ],
        scratch_shapes=[
            pltpu.VMEM((1, 256), jnp.float32),
            pltpu.VMEM((1, 256), jnp.float32),
        ],
    )(ks, qd, edge_attr, vs, we_w, we_b, be_w, be_b, aw2, exp16, woe, boe)


def _nodes1_body(acc_ref, op_ref, x_ref, ver_ref, exp_ref, woh_ref, boh_ref,
                 dc0_ref, dc1_ref, h_ref, st_ref, acc_sum, acc_sq):
    i = pl.program_id(0)
    s16 = acc_ref[:, 512:528] + op_ref[:, 0:16]
    rs16 = 1.0 / (s16 + 1e-30)
    rs256 = jnp.dot(rs16, exp_ref[...], preferred_element_type=jnp.float32)
    ld = jnp.log(s16[:, 8:9] + 1.0)
    wv = (acc_ref[:, 0:256]
          + jnp.dot(acc_ref[:, 256:512].astype(jnp.bfloat16), ver_ref[...],
                    preferred_element_type=jnp.float32))
    wv = wv * rs256
    hh = wv * dc0_ref[...] + (wv * ld) * dc1_ref[...]
    h = x_ref[...] + jnp.dot(hh.astype(jnp.bfloat16), woh_ref[...],
                             preferred_element_type=jnp.float32) + boh_ref[...]
    h_ref[...] = h

    @pl.when(i == 0)
    def _():
        acc_sum[...] = jnp.zeros_like(acc_sum)
        acc_sq[...] = jnp.zeros_like(acc_sq)

    acc_sum[...] += jnp.sum(h, axis=0, keepdims=True)
    acc_sq[...] += jnp.sum(h * h, axis=0, keepdims=True)

    @pl.when(i == pl.num_programs(0) - 1)
    def _():
        st_ref[0:1, :] = acc_sum[...]
        st_ref[1:2, :] = acc_sq[...]


def _tc_nodes1(acc, op, x, ver2, exp16, woh, boh, dc0, dc1, bn):
    n, d = x.shape
    grid = (n // bn,)
    return pl.pallas_call(
        _nodes1_body,
        grid=grid,
        in_specs=[
            pl.BlockSpec((bn, 640), lambda i: (i, 0)),
            pl.BlockSpec((bn, 128), lambda i: (i, 0)),
            pl.BlockSpec((bn, d), lambda i: (i, 0)),
            pl.BlockSpec((256, 256), lambda i: (0, 0)),
            pl.BlockSpec((16, 256), lambda i: (0, 0)),
            pl.BlockSpec((256, 256), lambda i: (0, 0)),
            pl.BlockSpec((1, 256), lambda i: (0, 0)),
            pl.BlockSpec((1, 256), lambda i: (0, 0)),
            pl.BlockSpec((1, 256), lambda i: (0, 0)),
        ],
        out_specs=[
            pl.BlockSpec((bn, 256), lambda i: (i, 0)),
            pl.BlockSpec((8, 256), lambda i: (0, 0)),
        ],
        out_shape=[
            jax.ShapeDtypeStruct((n, 256), jnp.float32),
            jax.ShapeDtypeStruct((8, 256), jnp.float32),
        ],
        scratch_shapes=[
            pltpu.VMEM((1, 256), jnp.float32),
            pltpu.VMEM((1, 256), jnp.float32),
        ],
    )(acc, op, x, ver2, exp16, woh, boh, dc0, dc1)


def _nodes2_body(h_ref, st_ref, w1_ref, b1_ref, w2_ref, b2_ref, g_ref,
                 bb_ref, cnt_ref, o_ref, st2_ref, acc_sum, acc_sq):
    i = pl.program_id(0)
    cnt = cnt_ref[0, 0]
    mean = st_ref[0:1, :] / cnt
    var = st_ref[1:2, :] / cnt - mean * mean
    hb = g_ref[...] * (h_ref[...] - mean) / jnp.sqrt(var + 1e-5) + bb_ref[...]
    r = jnp.maximum(jnp.dot(hb.astype(jnp.bfloat16), w1_ref[...],
                            preferred_element_type=jnp.float32) + b1_ref[...], 0.0)
    h2 = jnp.dot(r.astype(jnp.bfloat16), w2_ref[...],
                 preferred_element_type=jnp.float32) + b2_ref[...]
    h3 = hb + h2
    o_ref[...] = h3

    @pl.when(i == 0)
    def _():
        acc_sum[...] = jnp.zeros_like(acc_sum)
        acc_sq[...] = jnp.zeros_like(acc_sq)

    acc_sum[...] += jnp.sum(h3, axis=0, keepdims=True)
    acc_sq[...] += jnp.sum(h3 * h3, axis=0, keepdims=True)

    @pl.when(i == pl.num_programs(0) - 1)
    def _():
        st2_ref[0:1, :] = acc_sum[...]
        st2_ref[1:2, :] = acc_sq[...]


def _tc_nodes2(h_pre, stats, w1, b1, w2, b2, g, b, cnt, bn):
    n, d = h_pre.shape
    grid = (n // bn,)
    return pl.pallas_call(
        _nodes2_body,
        grid=grid,
        in_specs=[
            pl.BlockSpec((bn, d), lambda i: (i, 0)),
            pl.BlockSpec((8, 256), lambda i: (0, 0)),
            pl.BlockSpec((256, 512), lambda i: (0, 0)),
            pl.BlockSpec((1, 512), lambda i: (0, 0)),
            pl.BlockSpec((512, 256), lambda i: (0, 0)),
            pl.BlockSpec((1, 256), lambda i: (0, 0)),
            pl.BlockSpec((1, 256), lambda i: (0, 0)),
            pl.BlockSpec((1, 256), lambda i: (0, 0)),
            pl.BlockSpec((1, 1), lambda i: (0, 0), memory_space=pltpu.SMEM),
        ],
        out_specs=[
            pl.BlockSpec((bn, 256), lambda i: (i, 0)),
            pl.BlockSpec((8, 256), lambda i: (0, 0)),
        ],
        out_shape=[
            jax.ShapeDtypeStruct((n, 256), jnp.float32),
            jax.ShapeDtypeStruct((8, 256), jnp.float32),
        ],
        scratch_shapes=[
            pltpu.VMEM((1, 256), jnp.float32),
            pltpu.VMEM((1, 256), jnp.float32),
        ],
    )(h_pre, stats, w1, b1, w2, b2, g, b, cnt)


def _norm_body(v_ref, st_ref, g_ref, b_ref, cnt_ref, o_ref):
    cnt = cnt_ref[0, 0]
    mean = st_ref[0:1, :] / cnt
    var = st_ref[1:2, :] / cnt - mean * mean
    v = v_ref[...].astype(jnp.float32)
    o_ref[...] = (g_ref[...] * (v - mean)
                  / jnp.sqrt(var + 1e-5) + b_ref[...])


def _tc_norm(v, stats, g, b, cnt, bn):
    n, d = v.shape
    grid = (n // bn,)
    return pl.pallas_call(
        _norm_body,
        grid=grid,
        in_specs=[
            pl.BlockSpec((bn, d), lambda i: (i, 0)),
            pl.BlockSpec((8, 256), lambda i: (0, 0)),
            pl.BlockSpec((1, 256), lambda i: (0, 0)),
            pl.BlockSpec((1, 256), lambda i: (0, 0)),
            pl.BlockSpec((1, 1), lambda i: (0, 0), memory_space=pltpu.SMEM),
        ],
        out_specs=pl.BlockSpec((bn, d), lambda i: (i, 0)),
        out_shape=jax.ShapeDtypeStruct((n, d), jnp.float32),
    )(v, stats, g, b, cnt)


def _mesh():
    return plsc.VectorSubcoreMesh(core_axis_name="c", subcore_axis_name="s",
                                  num_cores=NC, num_subcores=NS)


def _sc_gather_kq_v(q, k, v, src3, dst3):
    n, d = k.shape
    nw = NC * NS
    nch = src3.shape[1]
    e = nw * nch * CH
    epw = nch * CH
    nfull = nch // GB
    tail = nch % GB
    rows_b = GB * CH

    def body(k_hbm, q_hbm, v_hbm, src_hbm, dst_hbm, ks_hbm, qd_hbm, vs_hbm,
             idx_s, idx_d, buf_a, buf_b, sem_a, sem_b):
        wid = lax.axis_index("s") * NC + lax.axis_index("c")
        pltpu.sync_copy(src_hbm.at[wid], idx_s)
        pltpu.sync_copy(dst_hbm.at[wid], idx_d)

        def run_table(tab_hbm, idx_v, out_hbm):
            def fire(bj, buf, sem, cnt):
                ds = []
                for kk in range(cnt):
                    ds.append(pltpu.async_copy(
                        tab_hbm.at[idx_v.at[bj * GB + kk, 0]],
                        buf.at[pl.ds(kk * CH, CH)], sem))
                return ds

            def drain(ds, buf, bj, cnt):
                for cp in ds:
                    cp.wait()
                pltpu.sync_copy(
                    buf.at[pl.ds(0, cnt * CH)],
                    out_hbm.at[pl.ds(wid * epw + bj * rows_b, cnt * CH)])

            def pair(t, carry):
                b0 = t * 2
                b1 = b0 + 1
                da = fire(b0, buf_a, sem_a, GB)
                db = fire(b1, buf_b, sem_b, GB)
                drain(da, buf_a, b0, GB)
                drain(db, buf_b, b1, GB)
                return carry

            lax.fori_loop(0, nfull // 2, pair, 0)
            if nfull % 2:
                bt = nfull - 1
                drain(fire(bt, buf_a, sem_a, GB), buf_a, bt, GB)
            if tail:
                drain(fire(nfull, buf_a, sem_a, tail), buf_a, nfull, tail)

        run_table(k_hbm, idx_s, ks_hbm)
        run_table(q_hbm, idx_d, qd_hbm)
        run_table(v_hbm, idx_s, vs_hbm)

    return pl.kernel(
        body,
        out_type=[
            jax.ShapeDtypeStruct((e, d), jnp.float32),
            jax.ShapeDtypeStruct((e, d), jnp.float32),
            jax.ShapeDtypeStruct((e, d), jnp.float32),
        ],
        mesh=_mesh(),
        scratch_types=[
            pltpu.VMEM((nch, 1, CH), jnp.int32),
            pltpu.VMEM((nch, 1, CH), jnp.int32),
            pltpu.VMEM((rows_b, d), jnp.float32),
            pltpu.VMEM((rows_b, d), jnp.float32),
            pltpu.SemaphoreType.DMA,
            pltpu.SemaphoreType.DMA,
        ],
    )(k, q, v, src3, dst3)


def _sc_scatter640(payload, dst3, dst3b, zeros128):
    n = zeros128.shape[0]
    nchb = dst3b.shape[1]
    nch = dst3.shape[1]
    eps = nchb * CHB
    epw = nch * CH
    rpw = n // NS

    def body(pay_hbm, dst_hbm, dstb_hbm, z_hbm, out_hbm, outp_hbm,
             idxb, idx4, pbuf_a, pbuf_b, accsh, sem_a, sem_b):
        c = lax.axis_index("c")
        s = lax.axis_index("s")
        rbase = s * rpw
        wid = c * NS + s
        pltpu.sync_copy(dstb_hbm.at[s], idxb)

        def run(col, idx_row, ch, nblk, ebase):
            def fire(bj, buf, sem):
                return pltpu.async_copy(
                    pay_hbm.at[pl.ds(ebase + bj * ch, ch), pl.ds(col, 128)],
                    buf.at[pl.ds(0, ch)], sem)

            def scat(bj, buf):
                pltpu.sync_copy(buf.at[pl.ds(0, ch)],
                                accsh.at[idx_row(bj)], add=True)

            def pair(t, carry):
                b0 = t * 2
                b1 = b0 + 1
                da = fire(b0, pbuf_a, sem_a)
                db = fire(b1, pbuf_b, sem_b)
                da.wait()
                scat(b0, pbuf_a)
                db.wait()
                scat(b1, pbuf_b)
                return carry

            lax.fori_loop(0, nblk // 2, pair, 0)
            if nblk % 2:
                bt = nblk - 1
                d = fire(bt, pbuf_a, sem_a)
                d.wait()
                scat(bt, pbuf_a)

        for f in range(2):
            col = (c * 2 + f) * 128
            pltpu.sync_copy(z_hbm.at[pl.ds(rbase, rpw)],
                            accsh.at[pl.ds(rbase, rpw)])
            plsc.subcore_barrier()
            run(col, lambda bj: idxb.at[bj, 0], CHB, nchb, s * eps)
            plsc.subcore_barrier()
            pltpu.sync_copy(accsh.at[pl.ds(rbase, rpw)],
                            out_hbm.at[pl.ds(rbase, rpw), pl.ds(col, 128)])
            plsc.subcore_barrier()

        pltpu.sync_copy(z_hbm.at[pl.ds(rbase, rpw)],
                        accsh.at[pl.ds(rbase, rpw)])
        plsc.subcore_barrier()
        def idx4_row(bj):
            pltpu.sync_copy(dst_hbm.at[wid, bj, 0], idx4)
            return idx4

        run(512, idx4_row, CH, nch, wid * epw)
        plsc.subcore_barrier()

        @pl.when(c == 0)
        def _():
            pltpu.sync_copy(accsh.at[pl.ds(rbase, rpw)],
                            out_hbm.at[pl.ds(rbase, rpw), pl.ds(512, 128)])

        @pl.when(c == 1)
        def _():
            pltpu.sync_copy(accsh.at[pl.ds(rbase, rpw)],
                            outp_hbm.at[pl.ds(rbase, rpw)])

    return pl.kernel(
        body,
        out_type=[
            jax.ShapeDtypeStruct((n, 640), jnp.float32),
            jax.ShapeDtypeStruct((n, 128), jnp.float32),
        ],
        mesh=_mesh(),
        scratch_types=[
            pltpu.VMEM((nchb, 1, CHB), jnp.int32),
            pltpu.VMEM((CH,), jnp.int32),
            pltpu.VMEM((CHB, 128), jnp.float32),
            pltpu.VMEM((CHB, 128), jnp.float32),
            pltpu.VMEM_SHARED((n, 128), jnp.float32),
            pltpu.SemaphoreType.DMA,
            pltpu.SemaphoreType.DMA,
        ],
    )(payload, dst3, dst3b, zeros128)


def kernel(x, edge_index, edge_attr, params):
    n, d = x.shape
    e = edge_attr.shape[0]
    src = edge_index[0]
    dst = edge_index[1]

    bn_blk = 1000 if n % 1000 == 0 else n // 10
    be_blk = 1600 if e % 1600 == 0 else e // 10

    wqkv = jnp.concatenate([params['Wq'], params['Wk'], params['Wv']], axis=1)
    bqkv = jnp.concatenate(
        [params['bq'], jnp.zeros((512,), jnp.float32)])[None, :]
    cw = np.arange(H)[:, None] * 64 + np.arange(DH)[None, :]
    cb = cw + DH
    we_w = params['We'][:, cw.reshape(-1)]
    we_b = params['We'][:, cb.reshape(-1)]
    be_w = params['be'][cw.reshape(-1)][None, :]
    be_b = params['be'][cb.reshape(-1)][None, :]
    awf = params['Aw'][:, :, 0].T.reshape(-1)
    eye16 = np.kron(np.eye(H, 16, dtype=np.float32), np.ones((DH, 1), np.float32))
    aw2 = awf[:, None] * eye16
    exp16_np = np.zeros((16, 256), np.float32)
    for hh in range(H):
        exp16_np[hh, hh * DH:(hh + 1) * DH] = 1.0
    exp16 = jnp.asarray(exp16_np)
    ver = params['VeRow']
    ver2 = jnp.zeros((256, 256), jnp.float32)
    for hh in range(H):
        ver2 = ver2.at[hh * DH:(hh + 1) * DH, hh * DH:(hh + 1) * DH].set(ver[:, hh, :])
    dc0 = params['deg_coef'][0, :, 0][None, :]
    dc1 = params['deg_coef'][0, :, 1][None, :]
    boh = params['boh'][None, :]
    boe = params['boe'][None, :]
    bf1 = params['bf1'][None, :]
    bf2 = params['bf2'][None, :]
    g1h = params['g1h'][None, :]
    b1h = params['b1h'][None, :]
    g1e = params['g1e'][None, :]
    b1e = params['b1e'][None, :]
    g2h = params['g2h'][None, :]
    b2h = params['b2h'][None, :]
    cnt_n = jnp.full((1, 1), float(n), jnp.float32)
    cnt_e = jnp.full((1, 1), float(e), jnp.float32)

    nw = NC * NS
    nch = e // (nw * CH)
    src3 = src.reshape(nw, nch, 1, CH)
    dst3 = dst.reshape(nw, nch, 1, CH)
    dst3b = dst.reshape(NS, e // (NS * CHB), 1, CHB)
    npad = ((n + 8 * NS - 1) // (8 * NS)) * (8 * NS)
    zeros128 = jnp.zeros((npad, 128), jnp.float32)

    bf = jnp.bfloat16
    wqkv = wqkv.astype(bf)
    we_w = we_w.astype(bf)
    we_b = we_b.astype(bf)
    aw2 = jnp.asarray(aw2).astype(bf)
    ver2 = ver2.astype(bf)
    woe_bf = params['Woe'].astype(bf)
    woh_bf = params['Woh'].astype(bf)
    w1_bf = params['W1'].astype(bf)
    w2_bf = params['W2'].astype(bf)

    q, k, v = _tc_nodeproj(x, wqkv, bqkv, bn_blk)
    ks, qd, vs = _sc_gather_kq_v(q, k, v, src3, dst3)
    payload, ee_pre, ee_stats = _tc_edges(
        ks, qd, edge_attr, vs, we_w, we_b, be_w, be_b, aw2, exp16,
        woe_bf, boe, be_blk)
    acc, accp = _sc_scatter640(payload, dst3, dst3b, zeros128)
    h_pre, h_stats = _tc_nodes1(acc, accp, x, ver2, exp16, woh_bf,
                                boh, dc0, dc1, bn_blk)
    h3, h_stats2 = _tc_nodes2(h_pre, h_stats, w1_bf, bf1,
                              w2_bf, bf2, g1h, b1h, cnt_n, bn_blk)
    h_out = _tc_norm(h3, h_stats2, g2h, b2h, cnt_n, bn_blk)
    ee_out = _tc_norm(ee_pre, ee_stats, g1e, b1e, cnt_e, be_blk)
    return h_out, ee_out


# speedup vs baseline: 2.6122x; 1.0081x over previous
import jax
import jax.numpy as jnp
import numpy as np
from jax import lax
from jax.experimental import pallas as pl
from jax.experimental.pallas import tpu as pltpu
from jax.experimental.pallas import tpu_sc as plsc

H = 8
DH = 32
NC = 2
NS = 16
CH = 40
CHB = 80
GB = 4


def _nodeproj_body(x_ref, w_ref, b_ref, q_ref, k_ref, v_ref):
    y = jnp.dot(x_ref[...].astype(jnp.bfloat16), w_ref[...],
                preferred_element_type=jnp.float32)
    y = y + b_ref[...]
    q_ref[...] = y[:, 0:256]
    k_ref[...] = y[:, 256:512]
    v_ref[...] = y[:, 512:768]


def _tc_nodeproj(x, wqkv, bqkv, bn):
    n, d = x.shape
    grid = (n // bn,)
    return pl.pallas_call(
        _nodeproj_body,
        grid=grid,
        in_specs=[
            pl.BlockSpec((bn, d), lambda i: (i, 0)),
            pl.BlockSpec((d, 768), lambda i: (0, 0)),
            pl.BlockSpec((1, 768), lambda i: (0, 0)),
        ],
        out_specs=[
            pl.BlockSpec((bn, 256), lambda i: (i, 0)),
            pl.BlockSpec((bn, 256), lambda i: (i, 0)),
            pl.BlockSpec((bn, 256), lambda i: (i, 0)),
        ],
        out_shape=[
            jax.ShapeDtypeStruct((n, 256), jnp.float32),
            jax.ShapeDtypeStruct((n, 256), jnp.float32),
            jax.ShapeDtypeStruct((n, 256), jnp.float32),
        ],
    )(x, wqkv, bqkv)


def _edges_body(ks_ref, qd_ref, ea_ref, vs_ref, ww_ref, wb_ref, bw_ref,
                bb_ref, aw_ref, exp_ref, woe_ref, boe_ref,
                pay_ref, ee_ref, st_ref, acc_sum, acc_sq):
    i = pl.program_id(0)
    ea = ea_ref[...]
    ea_bf = ea.astype(jnp.bfloat16)
    ew = jnp.dot(ea_bf, ww_ref[...], preferred_element_type=jnp.float32) + bw_ref[...]
    eb = jnp.dot(ea_bf, wb_ref[...], preferred_element_type=jnp.float32) + bb_ref[...]
    s = (ks_ref[...].astype(jnp.float32)
         + qd_ref[...].astype(jnp.float32)) * ew
    s = jnp.sign(s) * jnp.sqrt(jnp.abs(s))
    e_t = jnp.maximum(s + eb, 0.0)
    e_bf = e_t.astype(jnp.bfloat16)
    att = jnp.dot(e_bf, aw_ref[...], preferred_element_type=jnp.float32)
    att = jnp.clip(att, -5.0, 5.0)
    xa = jnp.exp(att)
    p256 = jnp.dot(xa, exp_ref[...], preferred_element_type=jnp.float32)
    pay_ref[:, 0:256] = vs_ref[...].astype(jnp.float32) * p256
    pay_ref[:, 256:512] = e_t * p256
    pay_ref[:, 512:528] = xa
    pay_ref[:, 528:640] = jnp.zeros((ea.shape[0], 112), jnp.float32)
    ee = ea + jnp.dot(e_bf, woe_ref[...],
                      preferred_element_type=jnp.float32) + boe_ref[...]
    ee_ref[...] = ee.astype(jnp.bfloat16)

    @pl.when(i == 0)
    def _():
        acc_sum[...] = jnp.zeros_like(acc_sum)
        acc_sq[...] = jnp.zeros_like(acc_sq)

    acc_sum[...] += jnp.sum(ee, axis=0, keepdims=True)
    acc_sq[...] += jnp.sum(ee * ee, axis=0, keepdims=True)

    @pl.when(i == pl.num_programs(0) - 1)
    def _():
        st_ref[0:1, :] = acc_sum[...]
        st_ref[1:2, :] = acc_sq[...]


def _tc_edges(ks, qd, edge_attr, vs, we_w, we_b, be_w, be_b, aw2, exp16,
              woe, boe, be_blk):
    e, d = edge_attr.shape
    grid = (e // be_blk,)
    return pl.pallas_call(
        _edges_body,
        grid=grid,
        in_specs=[
            pl.BlockSpec((be_blk, 256), lambda i: (i, 0)),
            pl.BlockSpec((be_blk, 256), lambda i: (i, 0)),
            pl.BlockSpec((be_blk, d), lambda i: (i, 0)),
            pl.BlockSpec((be_blk, 256), lambda i: (i, 0)),
            pl.BlockSpec((d, 256), lambda i: (0, 0)),
            pl.BlockSpec((d, 256), lambda i: (0, 0)),
            pl.BlockSpec((1, 256), lambda i: (0, 0)),
            pl.BlockSpec((1, 256), lambda i: (0, 0)),
            pl.BlockSpec((256, 16), lambda i: (0, 0)),
            pl.BlockSpec((16, 256), lambda i: (0, 0)),
            pl.BlockSpec((256, 256), lambda i: (0, 0)),
            pl.BlockSpec((1, 256), lambda i: (0, 0)),
        ],
        out_specs=[
            pl.BlockSpec((be_blk, 640), lambda i: (i, 0)),
            pl.BlockSpec((be_blk, 256), lambda i: (i, 0)),
            pl.BlockSpec((8, 256), lambda i: (0, 0)),
        ],
        out_shape=[
            jax.ShapeDtypeStruct((e, 640), jnp.float32),
            jax.ShapeDtypeStruct((e, 256), jnp.bfloat16),
            jax.ShapeDtypeStruct((8, 256), jnp.float32),
        ],
        scratch_shapes=[
            pltpu.VMEM((1, 256), jnp.float32),
            pltpu.VMEM((1, 256), jnp.float32),
        ],
    )(ks, qd, edge_attr, vs, we_w, we_b, be_w, be_b, aw2, exp16, woe, boe)


def _nodes1_body(acc_ref, op_ref, x_ref, ver_ref, exp_ref, woh_ref, boh_ref,
                 dc0_ref, dc1_ref, h_ref, st_ref, acc_sum, acc_sq):
    i = pl.program_id(0)
    s16 = acc_ref[:, 512:528] + op_ref[:, 0:16]
    rs16 = 1.0 / (s16 + 1e-30)
    rs256 = jnp.dot(rs16, exp_ref[...], preferred_element_type=jnp.float32)
    ld = jnp.log(s16[:, 8:9] + 1.0)
    wv = (acc_ref[:, 0:256]
          + jnp.dot(acc_ref[:, 256:512].astype(jnp.bfloat16), ver_ref[...],
                    preferred_element_type=jnp.float32))
    wv = wv * rs256
    hh = wv * dc0_ref[...] + (wv * ld) * dc1_ref[...]
    h = x_ref[...] + jnp.dot(hh.astype(jnp.bfloat16), woh_ref[...],
                             preferred_element_type=jnp.float32) + boh_ref[...]
    h_ref[...] = h

    @pl.when(i == 0)
    def _():
        acc_sum[...] = jnp.zeros_like(acc_sum)
        acc_sq[...] = jnp.zeros_like(acc_sq)

    acc_sum[...] += jnp.sum(h, axis=0, keepdims=True)
    acc_sq[...] += jnp.sum(h * h, axis=0, keepdims=True)

    @pl.when(i == pl.num_programs(0) - 1)
    def _():
        st_ref[0:1, :] = acc_sum[...]
        st_ref[1:2, :] = acc_sq[...]


def _tc_nodes1(acc, op, x, ver2, exp16, woh, boh, dc0, dc1, bn):
    n, d = x.shape
    grid = (n // bn,)
    return pl.pallas_call(
        _nodes1_body,
        grid=grid,
        in_specs=[
            pl.BlockSpec((bn, 640), lambda i: (i, 0)),
            pl.BlockSpec((bn, 128), lambda i: (i, 0)),
            pl.BlockSpec((bn, d), lambda i: (i, 0)),
            pl.BlockSpec((256, 256), lambda i: (0, 0)),
            pl.BlockSpec((16, 256), lambda i: (0, 0)),
            pl.BlockSpec((256, 256), lambda i: (0, 0)),
            pl.BlockSpec((1, 256), lambda i: (0, 0)),
            pl.BlockSpec((1, 256), lambda i: (0, 0)),
            pl.BlockSpec((1, 256), lambda i: (0, 0)),
        ],
        out_specs=[
            pl.BlockSpec((bn, 256), lambda i: (i, 0)),
            pl.BlockSpec((8, 256), lambda i: (0, 0)),
        ],
        out_shape=[
            jax.ShapeDtypeStruct((n, 256), jnp.float32),
            jax.ShapeDtypeStruct((8, 256), jnp.float32),
        ],
        scratch_shapes=[
            pltpu.VMEM((1, 256), jnp.float32),
            pltpu.VMEM((1, 256), jnp.float32),
        ],
    )(acc, op, x, ver2, exp16, woh, boh, dc0, dc1)


def _nodes2_body(h_ref, st_ref, w1_ref, b1_ref, w2_ref, b2_ref, g_ref,
                 bb_ref, cnt_ref, o_ref, st2_ref, acc_sum, acc_sq):
    i = pl.program_id(0)
    cnt = cnt_ref[0, 0]
    mean = st_ref[0:1, :] / cnt
    var = st_ref[1:2, :] / cnt - mean * mean
    hb = g_ref[...] * (h_ref[...] - mean) / jnp.sqrt(var + 1e-5) + bb_ref[...]
    r = jnp.maximum(jnp.dot(hb.astype(jnp.bfloat16), w1_ref[...],
                            preferred_element_type=jnp.float32) + b1_ref[...], 0.0)
    h2 = jnp.dot(r.astype(jnp.bfloat16), w2_ref[...],
                 preferred_element_type=jnp.float32) + b2_ref[...]
    h3 = hb + h2
    o_ref[...] = h3

    @pl.when(i == 0)
    def _():
        acc_sum[...] = jnp.zeros_like(acc_sum)
        acc_sq[...] = jnp.zeros_like(acc_sq)

    acc_sum[...] += jnp.sum(h3, axis=0, keepdims=True)
    acc_sq[...] += jnp.sum(h3 * h3, axis=0, keepdims=True)

    @pl.when(i == pl.num_programs(0) - 1)
    def _():
        st2_ref[0:1, :] = acc_sum[...]
        st2_ref[1:2, :] = acc_sq[...]


def _tc_nodes2(h_pre, stats, w1, b1, w2, b2, g, b, cnt, bn):
    n, d = h_pre.shape
    grid = (n // bn,)
    return pl.pallas_call(
        _nodes2_body,
        grid=grid,
        in_specs=[
            pl.BlockSpec((bn, d), lambda i: (i, 0)),
            pl.BlockSpec((8, 256), lambda i: (0, 0)),
            pl.BlockSpec((256, 512), lambda i: (0, 0)),
            pl.BlockSpec((1, 512), lambda i: (0, 0)),
            pl.BlockSpec((512, 256), lambda i: (0, 0)),
            pl.BlockSpec((1, 256), lambda i: (0, 0)),
            pl.BlockSpec((1, 256), lambda i: (0, 0)),
            pl.BlockSpec((1, 256), lambda i: (0, 0)),
            pl.BlockSpec((1, 1), lambda i: (0, 0), memory_space=pltpu.SMEM),
        ],
        out_specs=[
            pl.BlockSpec((bn, 256), lambda i: (i, 0)),
            pl.BlockSpec((8, 256), lambda i: (0, 0)),
        ],
        out_shape=[
            jax.ShapeDtypeStruct((n, 256), jnp.float32),
            jax.ShapeDtypeStruct((8, 256), jnp.float32),
        ],
        scratch_shapes=[
            pltpu.VMEM((1, 256), jnp.float32),
            pltpu.VMEM((1, 256), jnp.float32),
        ],
    )(h_pre, stats, w1, b1, w2, b2, g, b, cnt)


def _norm_body(v_ref, st_ref, g_ref, b_ref, cnt_ref, o_ref):
    cnt = cnt_ref[0, 0]
    mean = st_ref[0:1, :] / cnt
    var = st_ref[1:2, :] / cnt - mean * mean
    v = v_ref[...].astype(jnp.float32)
    o_ref[...] = (g_ref[...] * (v - mean)
                  / jnp.sqrt(var + 1e-5) + b_ref[...])


def _tc_norm(v, stats, g, b, cnt, bn):
    n, d = v.shape
    grid = (n // bn,)
    return pl.pallas_call(
        _norm_body,
        grid=grid,
        in_specs=[
            pl.BlockSpec((bn, d), lambda i: (i, 0)),
            pl.BlockSpec((8, 256), lambda i: (0, 0)),
            pl.BlockSpec((1, 256), lambda i: (0, 0)),
            pl.BlockSpec((1, 256), lambda i: (0, 0)),
            pl.BlockSpec((1, 1), lambda i: (0, 0), memory_space=pltpu.SMEM),
        ],
        out_specs=pl.BlockSpec((bn, d), lambda i: (i, 0)),
        out_shape=jax.ShapeDtypeStruct((n, d), jnp.float32),
    )(v, stats, g, b, cnt)


def _mesh():
    return plsc.VectorSubcoreMesh(core_axis_name="c", subcore_axis_name="s",
                                  num_cores=NC, num_subcores=NS)


def _sc_gather_kq_v(q, k, v, src3, dst3):
    n, d = k.shape
    nw = NC * NS
    nch = src3.shape[1]
    e = nw * nch * CH
    epw = nch * CH
    nfull = nch // GB
    tail = nch % GB
    rows_b = GB * CH

    def body(k_hbm, q_hbm, v_hbm, src_hbm, dst_hbm, ks_hbm, qd_hbm, vs_hbm,
             idx_s, idx_d, buf_a, buf_b, sem_a, sem_b):
        wid = lax.axis_index("s") * NC + lax.axis_index("c")
        pltpu.sync_copy(src_hbm.at[wid], idx_s)
        pltpu.sync_copy(dst_hbm.at[wid], idx_d)

        def run_table(tab_hbm, idx_v, out_hbm):
            def fire(bj, buf, sem, cnt):
                ds = []
                for kk in range(cnt):
                    ds.append(pltpu.async_copy(
                        tab_hbm.at[idx_v.at[bj * GB + kk, 0]],
                        buf.at[pl.ds(kk * CH, CH)], sem))
                return ds

            def drain(ds, buf, bj, cnt):
                for cp in ds:
                    cp.wait()
                pltpu.sync_copy(
                    buf.at[pl.ds(0, cnt * CH)],
                    out_hbm.at[pl.ds(wid * epw + bj * rows_b, cnt * CH)])

            def pair(t, carry):
                b0 = t * 2
                b1 = b0 + 1
                da = fire(b0, buf_a, sem_a, GB)
                db = fire(b1, buf_b, sem_b, GB)
                drain(da, buf_a, b0, GB)
                drain(db, buf_b, b1, GB)
                return carry

            lax.fori_loop(0, nfull // 2, pair, 0)
            if nfull % 2:
                bt = nfull - 1
                drain(fire(bt, buf_a, sem_a, GB), buf_a, bt, GB)
            if tail:
                drain(fire(nfull, buf_a, sem_a, tail), buf_a, nfull, tail)

        run_table(k_hbm, idx_s, ks_hbm)
        run_table(q_hbm, idx_d, qd_hbm)
        run_table(v_hbm, idx_s, vs_hbm)

    return pl.kernel(
        body,
        out_type=[
            jax.ShapeDtypeStruct((e, d), jnp.float32),
            jax.ShapeDtypeStruct((e, d), jnp.float32),
            jax.ShapeDtypeStruct((e, d), jnp.float32),
        ],
        mesh=_mesh(),
        scratch_types=[
            pltpu.VMEM((nch, 1, CH), jnp.int32),
            pltpu.VMEM((nch, 1, CH), jnp.int32),
            pltpu.VMEM((rows_b, d), jnp.float32),
            pltpu.VMEM((rows_b, d), jnp.float32),
            pltpu.SemaphoreType.DMA,
            pltpu.SemaphoreType.DMA,
        ],
    )(k, q, v, src3, dst3)


def _sc_scatter640(payload, dst3, dst3b, zeros128):
    n = zeros128.shape[0]
    nchb = dst3b.shape[1]
    nch = dst3.shape[1]
    eps = nchb * CHB
    epw = nch * CH
    rpw = n // NS

    def body(pay_hbm, dst_hbm, dstb_hbm, z_hbm, out_hbm, outp_hbm,
             idxb, idx4, pbuf_a, pbuf_b, accsh, sem_a, sem_b):
        c = lax.axis_index("c")
        s = lax.axis_index("s")
        rbase = s * rpw
        wid = c * NS + s
        pltpu.sync_copy(dstb_hbm.at[s], idxb)

        def run(col, idx_row, ch, nblk, ebase):
            def fire(bj, buf, sem):
                return pltpu.async_copy(
                    pay_hbm.at[pl.ds(ebase + bj * ch, ch), pl.ds(col, 128)],
                    buf.at[pl.ds(0, ch)], sem)

            def scat(bj, buf):
                pltpu.sync_copy(buf.at[pl.ds(0, ch)],
                                accsh.at[idx_row(bj)], add=True)

            def pair(t, carry):
                b0 = t * 2
                b1 = b0 + 1
                da = fire(b0, pbuf_a, sem_a)
                db = fire(b1, pbuf_b, sem_b)
                da.wait()
                scat(b0, pbuf_a)
                db.wait()
                scat(b1, pbuf_b)
                return carry

            lax.fori_loop(0, nblk // 2, pair, 0)
            if nblk % 2:
                bt = nblk - 1
                d = fire(bt, pbuf_a, sem_a)
                d.wait()
                scat(bt, pbuf_a)

        for f in range(2):
            col = (c * 2 + f) * 128
            pltpu.sync_copy(z_hbm.at[pl.ds(rbase, rpw)],
                            accsh.at[pl.ds(rbase, rpw)])
            plsc.subcore_barrier()
            run(col, lambda bj: idxb.at[bj, 0], CHB, nchb, s * eps)
            plsc.subcore_barrier()
            pltpu.sync_copy(accsh.at[pl.ds(rbase, rpw)],
                            out_hbm.at[pl.ds(rbase, rpw), pl.ds(col, 128)])
            plsc.subcore_barrier()

        pltpu.sync_copy(z_hbm.at[pl.ds(rbase, rpw)],
                        accsh.at[pl.ds(rbase, rpw)])
        plsc.subcore_barrier()
        def idx4_row(bj):
            pltpu.sync_copy(dst_hbm.at[wid, bj, 0], idx4)
            return idx4

        run(512, idx4_row, CH, nch, wid * epw)
        plsc.subcore_barrier()

        @pl.when(c == 0)
        def _():
            pltpu.sync_copy(accsh.at[pl.ds(rbase, rpw)],
                            out_hbm.at[pl.ds(rbase, rpw), pl.ds(512, 128)])

        @pl.when(c == 1)
        def _():
            pltpu.sync_copy(accsh.at[pl.ds(rbase, rpw)],
                            outp_hbm.at[pl.ds(rbase, rpw)])

    return pl.kernel(
        body,
        out_type=[
            jax.ShapeDtypeStruct((n, 640), jnp.float32),
            jax.ShapeDtypeStruct((n, 128), jnp.float32),
        ],
        mesh=_mesh(),
        scratch_types=[
            pltpu.VMEM((nchb, 1, CHB), jnp.int32),
            pltpu.VMEM((CH,), jnp.int32),
            pltpu.VMEM((CHB, 128), jnp.float32),
            pltpu.VMEM((CHB, 128), jnp.float32),
            pltpu.VMEM_SHARED((n, 128), jnp.float32),
            pltpu.SemaphoreType.DMA,
            pltpu.SemaphoreType.DMA,
        ],
    )(payload, dst3, dst3b, zeros128)


def kernel(x, edge_index, edge_attr, params):
    n, d = x.shape
    e = edge_attr.shape[0]
    src = edge_index[0]
    dst = edge_index[1]

    bn_blk = 2000 if n % 2000 == 0 else n // 10
    be_blk = 3200 if e % 3200 == 0 else e // 10

    wqkv = jnp.concatenate([params['Wq'], params['Wk'], params['Wv']], axis=1)
    bqkv = jnp.concatenate(
        [params['bq'], jnp.zeros((512,), jnp.float32)])[None, :]
    cw = np.arange(H)[:, None] * 64 + np.arange(DH)[None, :]
    cb = cw + DH
    we_w = params['We'][:, cw.reshape(-1)]
    we_b = params['We'][:, cb.reshape(-1)]
    be_w = params['be'][cw.reshape(-1)][None, :]
    be_b = params['be'][cb.reshape(-1)][None, :]
    awf = params['Aw'][:, :, 0].T.reshape(-1)
    eye16 = np.kron(np.eye(H, 16, dtype=np.float32), np.ones((DH, 1), np.float32))
    aw2 = awf[:, None] * eye16
    exp16_np = np.zeros((16, 256), np.float32)
    for hh in range(H):
        exp16_np[hh, hh * DH:(hh + 1) * DH] = 1.0
    exp16 = jnp.asarray(exp16_np)
    ver = params['VeRow']
    ver2 = jnp.zeros((256, 256), jnp.float32)
    for hh in range(H):
        ver2 = ver2.at[hh * DH:(hh + 1) * DH, hh * DH:(hh + 1) * DH].set(ver[:, hh, :])
    dc0 = params['deg_coef'][0, :, 0][None, :]
    dc1 = params['deg_coef'][0, :, 1][None, :]
    boh = params['boh'][None, :]
    boe = params['boe'][None, :]
    bf1 = params['bf1'][None, :]
    bf2 = params['bf2'][None, :]
    g1h = params['g1h'][None, :]
    b1h = params['b1h'][None, :]
    g1e = params['g1e'][None, :]
    b1e = params['b1e'][None, :]
    g2h = params['g2h'][None, :]
    b2h = params['b2h'][None, :]
    cnt_n = jnp.full((1, 1), float(n), jnp.float32)
    cnt_e = jnp.full((1, 1), float(e), jnp.float32)

    nw = NC * NS
    nch = e // (nw * CH)
    src3 = src.reshape(nw, nch, 1, CH)
    dst3 = dst.reshape(nw, nch, 1, CH)
    dst3b = dst.reshape(NS, e // (NS * CHB), 1, CHB)
    npad = ((n + 8 * NS - 1) // (8 * NS)) * (8 * NS)
    zeros128 = jnp.zeros((npad, 128), jnp.float32)

    bf = jnp.bfloat16
    wqkv = wqkv.astype(bf)
    we_w = we_w.astype(bf)
    we_b = we_b.astype(bf)
    aw2 = jnp.asarray(aw2).astype(bf)
    ver2 = ver2.astype(bf)
    woe_bf = params['Woe'].astype(bf)
    woh_bf = params['Woh'].astype(bf)
    w1_bf = params['W1'].astype(bf)
    w2_bf = params['W2'].astype(bf)

    q, k, v = _tc_nodeproj(x, wqkv, bqkv, bn_blk)
    ks, qd, vs = _sc_gather_kq_v(q, k, v, src3, dst3)
    payload, ee_pre, ee_stats = _tc_edges(
        ks, qd, edge_attr, vs, we_w, we_b, be_w, be_b, aw2, exp16,
        woe_bf, boe, be_blk)
    acc, accp = _sc_scatter640(payload, dst3, dst3b, zeros128)
    h_pre, h_stats = _tc_nodes1(acc, accp, x, ver2, exp16, woh_bf,
                                boh, dc0, dc1, bn_blk)
    h3, h_stats2 = _tc_nodes2(h_pre, h_stats, w1_bf, bf1,
                              w2_bf, bf2, g1h, b1h, cnt_n, bn_blk)
    h_out = _tc_norm(h3, h_stats2, g2h, b2h, cnt_n, bn_blk)
    ee_out = _tc_norm(ee_pre, ee_stats, g1e, b1e, cnt_e, be_blk)
    return h_out, ee_out
